# Initial kernel scaffold; baseline (speedup 1.0000x reference)
import jax
import jax.numpy as jnp
from jax.experimental import pallas as pl


def kernel(x, edge_index, W1, b1, W2, b2):
    raise NotImplementedError("write your pallas kernel here")


# speedup vs baseline: 12.1353x; 12.1353x over previous
import functools

import jax
import jax.numpy as jnp
from jax import lax
from jax.experimental import pallas as pl
from jax.experimental.pallas import tpu as pltpu
from jax.experimental.pallas import tpu_sc as plsc

N_NODES = 10000
N_EDGES = 320000
D = 128

NC = 2
NS = 16
NW = NC * NS
NPAD = 10240
ROWS = NPAD // NS
EPT = N_EDGES // NW
K = 80
ITERS = EPT // K

_mesh = plsc.VectorSubcoreMesh(core_axis_name="c", subcore_axis_name="s")


def _deg_body(dst3, ones_hbm, z128, out_hbm, idx_v, ones_v, shared):
    c = lax.axis_index("c")
    s = lax.axis_index("s")
    w = c * NS + s
    pltpu.sync_copy(ones_hbm, ones_v)
    pltpu.sync_copy(z128, shared.at[pl.ds(s * ROWS, ROWS)])
    plsc.subcore_barrier()

    def step(i, carry):
        pltpu.sync_copy(dst3.at[w, i], idx_v)
        pltpu.sync_copy(ones_v, shared.at[idx_v], add=True)
        return carry

    lax.fori_loop(0, ITERS, step, 0)
    plsc.subcore_barrier()
    pltpu.sync_copy(shared.at[pl.ds(s * ROWS, ROWS)],
                    out_hbm.at[pl.ds(c * NPAD + s * ROWS, ROWS)])


_deg_call = pl.kernel(
    _deg_body,
    out_type=jax.ShapeDtypeStruct((NC * NPAD, D), jnp.float32),
    mesh=_mesh,
    scratch_types=[
        pltpu.VMEM((K,), jnp.int32),
        pltpu.VMEM((K, D), jnp.float32),
        pltpu.VMEM_SHARED((NPAD, D), jnp.float32),
    ],
)


def _edge_body(src3, dst3, y_hbm, z128, out_hbm, sidx_v, didx_v, rows_v, sem,
               shared):
    c = lax.axis_index("c")
    s = lax.axis_index("s")
    w = c * NS + s
    pltpu.sync_copy(z128, shared.at[pl.ds(s * ROWS, ROWS)])
    plsc.subcore_barrier()

    def step(i, carry):
        pltpu.sync_copy(src3.at[w, i], sidx_v)
        pltpu.sync_copy(dst3.at[w, i], didx_v)
        pltpu.async_copy(y_hbm.at[sidx_v], rows_v, sem).wait()
        pltpu.sync_copy(rows_v, shared.at[didx_v], add=True)
        return carry

    lax.fori_loop(0, ITERS, step, 0)
    plsc.subcore_barrier()
    pltpu.sync_copy(shared.at[pl.ds(s * ROWS, ROWS)],
                    out_hbm.at[pl.ds(c * NPAD + s * ROWS, ROWS)])


_edge_call = pl.kernel(
    _edge_body,
    out_type=jax.ShapeDtypeStruct((NC * NPAD, D), jnp.float32),
    mesh=_mesh,
    scratch_types=[
        pltpu.VMEM((K,), jnp.int32),
        pltpu.VMEM((K,), jnp.int32),
        pltpu.VMEM((K, D), jnp.float32),
        pltpu.SemaphoreType.DMA,
        pltpu.VMEM_SHARED((NPAD, D), jnp.float32),
    ],
)

BM = 1000
GRID = N_NODES // BM


def _dinv(dp_ref):
    return lax.rsqrt(1.0 + dp_ref[0, :, 0] + dp_ref[1, :, 0])


def _mm_scale_body(dp_ref, x_ref, w_ref, o_ref):
    dinv = _dinv(dp_ref)
    o_ref[...] = jnp.dot(x_ref[...], w_ref[...],
                         preferred_element_type=jnp.float32) * dinv[:, None]


def _mid_body(dp_ref, a_ref, y_ref, b_ref, w_ref, o_ref):
    dinv = _dinv(dp_ref)
    h = jnp.maximum(
        (a_ref[0] + a_ref[1] + y_ref[...]) * dinv[:, None] + b_ref[...], 0.0)
    o_ref[...] = jnp.dot(h, w_ref[...],
                         preferred_element_type=jnp.float32) * dinv[:, None]


def _out_body(dp_ref, a_ref, y_ref, b_ref, o_ref):
    dinv = _dinv(dp_ref)
    o_ref[...] = jnp.maximum(
        (a_ref[0] + a_ref[1] + y_ref[...]) * dinv[:, None] + b_ref[...], 0.0)


_dp_spec = pl.BlockSpec((2, BM, D), lambda i: (0, i, 0))
_a_spec = pl.BlockSpec((2, BM, D), lambda i: (0, i, 0))
_row_spec = pl.BlockSpec((BM, D), lambda i: (i, 0))
_w_spec = pl.BlockSpec((D, D), lambda i: (0, 0))
_b_spec = pl.BlockSpec((1, D), lambda i: (0, 0))
_o_shape = jax.ShapeDtypeStruct((N_NODES, D), jnp.float32)

_mm_scale = pl.pallas_call(
    _mm_scale_body, grid=(GRID,),
    in_specs=[_dp_spec, _row_spec, _w_spec],
    out_specs=_row_spec, out_shape=_o_shape)

_mid = pl.pallas_call(
    _mid_body, grid=(GRID,),
    in_specs=[_dp_spec, _a_spec, _row_spec, _b_spec, _w_spec],
    out_specs=_row_spec, out_shape=_o_shape)

_out = pl.pallas_call(
    _out_body, grid=(GRID,),
    in_specs=[_dp_spec, _a_spec, _row_spec, _b_spec],
    out_specs=_row_spec, out_shape=_o_shape)


@jax.jit
def kernel(x, edge_index, W1, b1, W2, b2):
    src3 = edge_index[0].reshape(NW, ITERS, K)
    dst3 = edge_index[1].reshape(NW, ITERS, K)
    ones128 = jnp.ones((K, D), jnp.float32)
    z128 = jnp.zeros((ROWS, D), jnp.float32)

    dp = _deg_call(dst3, ones128, z128).reshape(NC, NPAD, D)
    y1 = _mm_scale(dp, x, W1)
    a1 = _edge_call(src3, dst3, y1, z128).reshape(NC, NPAD, D)
    y2 = _mid(dp, a1, y1, b1.reshape(1, D), W2)
    a2 = _edge_call(src3, dst3, y2, z128).reshape(NC, NPAD, D)
    return _out(dp, a2, y2, b2.reshape(1, D))


# speedup vs baseline: 19.0941x; 1.5734x over previous
import functools

import jax
import jax.numpy as jnp
from jax import lax
from jax.experimental import pallas as pl
from jax.experimental.pallas import tpu as pltpu
from jax.experimental.pallas import tpu_sc as plsc

N_NODES = 10000
N_EDGES = 320000
D = 128

NC = 2
NS = 16
NW = NC * NS
NPAD = 10240
ROWS = NPAD // NS
EPT = N_EDGES // NW
K = 80
ITERS = EPT // K

_mesh = plsc.VectorSubcoreMesh(core_axis_name="c", subcore_axis_name="s")


NBUF = 5
DGROUPS = ITERS // NBUF


def _deg_body(dst2, ones_hbm, z128, out_hbm, i0, i1, i2, i3, i4, ones_v, s0,
              s1, s2, s3, s4, shared):
    c = lax.axis_index("c")
    s = lax.axis_index("s")
    w = c * NS + s
    idx = [i0, i1, i2, i3, i4]
    sems = [s0, s1, s2, s3, s4]
    pltpu.sync_copy(ones_hbm, ones_v)
    pltpu.sync_copy(z128, shared.at[pl.ds(s * ROWS, ROWS)])
    plsc.subcore_barrier()

    for b in range(NBUF):
        pltpu.sync_copy(dst2.at[w * ITERS + b], idx[b])
        pltpu.async_copy(ones_v, shared.at[idx[b]], sems[b], add=True)

    def group(gi, carry):
        for b in range(NBUF):
            nxt = gi * NBUF + b + NBUF
            pltpu.make_async_copy(ones_v, shared.at[idx[b]], sems[b]).wait()
            pltpu.sync_copy(dst2.at[w * ITERS + nxt], idx[b])
            pltpu.async_copy(ones_v, shared.at[idx[b]], sems[b], add=True)
        return carry

    lax.fori_loop(0, DGROUPS - 1, group, 0)
    for b in range(NBUF):
        pltpu.make_async_copy(ones_v, shared.at[idx[b]], sems[b]).wait()
    plsc.subcore_barrier()
    pltpu.sync_copy(shared.at[pl.ds(s * ROWS, ROWS)],
                    out_hbm.at[pl.ds(c * NPAD + s * ROWS, ROWS)])


_deg_call = pl.kernel(
    _deg_body,
    out_type=jax.ShapeDtypeStruct((NC * NPAD, D), jnp.float32),
    mesh=_mesh,
    scratch_types=(
        [pltpu.VMEM((K,), jnp.int32)] * NBUF
        + [pltpu.VMEM((K, D), jnp.float32)]
        + [pltpu.SemaphoreType.DMA] * NBUF
        + [pltpu.VMEM_SHARED((NPAD, D), jnp.float32)]
    ),
)


def _edge_body(src2, dst2, y_hbm, z128, out_hbm, si0, si1, di0, di1, r0, r1,
               g0, g1, shared):
    c = lax.axis_index("c")
    s = lax.axis_index("s")
    w = c * NS + s
    sidx = [si0, si1]
    didx = [di0, di1]
    rows = [r0, r1]
    gsem = [g0, g1]
    pltpu.sync_copy(z128, shared.at[pl.ds(s * ROWS, ROWS)])
    plsc.subcore_barrier()

    for b in range(2):
        pltpu.sync_copy(src2.at[w * ITERS + b], sidx[b])
        pltpu.sync_copy(dst2.at[w * ITERS + b], didx[b])
        pltpu.async_copy(y_hbm.at[sidx[b]], rows[b], gsem[b])

    def group(gi, carry):
        for b in range(2):
            nxt = gi * 2 + b + 2
            pltpu.make_async_copy(y_hbm.at[sidx[b]], rows[b], gsem[b]).wait()
            pltpu.sync_copy(rows[b], shared.at[didx[b]], add=True)
            pltpu.sync_copy(src2.at[w * ITERS + nxt], sidx[b])
            pltpu.sync_copy(dst2.at[w * ITERS + nxt], didx[b])
            pltpu.async_copy(y_hbm.at[sidx[b]], rows[b], gsem[b])
        return carry

    lax.fori_loop(0, (ITERS - 3) // 2, group, 0)
    pltpu.make_async_copy(y_hbm.at[sidx[0]], rows[0], gsem[0]).wait()
    pltpu.sync_copy(rows[0], shared.at[didx[0]], add=True)
    pltpu.sync_copy(src2.at[w * ITERS + ITERS - 1], sidx[0])
    pltpu.sync_copy(dst2.at[w * ITERS + ITERS - 1], didx[0])
    pltpu.async_copy(y_hbm.at[sidx[0]], rows[0], gsem[0])
    pltpu.make_async_copy(y_hbm.at[sidx[1]], rows[1], gsem[1]).wait()
    pltpu.sync_copy(rows[1], shared.at[didx[1]], add=True)
    pltpu.make_async_copy(y_hbm.at[sidx[0]], rows[0], gsem[0]).wait()
    pltpu.sync_copy(rows[0], shared.at[didx[0]], add=True)

    plsc.subcore_barrier()
    pltpu.sync_copy(shared.at[pl.ds(s * ROWS, ROWS)],
                    out_hbm.at[pl.ds(c * NPAD + s * ROWS, ROWS)])


_edge_call = pl.kernel(
    _edge_body,
    out_type=jax.ShapeDtypeStruct((NC * NPAD, D), jnp.float32),
    mesh=_mesh,
    scratch_types=(
        [pltpu.VMEM((K,), jnp.int32)] * 2
        + [pltpu.VMEM((K,), jnp.int32)] * 2
        + [pltpu.VMEM((K, D), jnp.float32)] * 2
        + [pltpu.SemaphoreType.DMA] * 2
        + [pltpu.VMEM_SHARED((NPAD, D), jnp.float32)]
    ),
)

BM = 1000
GRID = N_NODES // BM


def _dinv(dp_ref):
    return lax.rsqrt(1.0 + dp_ref[0, :, 0] + dp_ref[1, :, 0])


def _mm_scale_body(dp_ref, x_ref, w_ref, o_ref):
    dinv = _dinv(dp_ref)
    o_ref[...] = jnp.dot(x_ref[...], w_ref[...],
                         preferred_element_type=jnp.float32) * dinv[:, None]


def _mid_body(dp_ref, a_ref, y_ref, b_ref, w_ref, o_ref):
    dinv = _dinv(dp_ref)
    h = jnp.maximum(
        (a_ref[0] + a_ref[1] + y_ref[...]) * dinv[:, None] + b_ref[...], 0.0)
    o_ref[...] = jnp.dot(h, w_ref[...],
                         preferred_element_type=jnp.float32) * dinv[:, None]


def _out_body(dp_ref, a_ref, y_ref, b_ref, o_ref):
    dinv = _dinv(dp_ref)
    o_ref[...] = jnp.maximum(
        (a_ref[0] + a_ref[1] + y_ref[...]) * dinv[:, None] + b_ref[...], 0.0)


_dp_spec = pl.BlockSpec((2, BM, D), lambda i: (0, i, 0))
_a_spec = pl.BlockSpec((2, BM, D), lambda i: (0, i, 0))
_row_spec = pl.BlockSpec((BM, D), lambda i: (i, 0))
_w_spec = pl.BlockSpec((D, D), lambda i: (0, 0))
_b_spec = pl.BlockSpec((1, D), lambda i: (0, 0))
_o_shape = jax.ShapeDtypeStruct((N_NODES, D), jnp.float32)

_mm_scale = pl.pallas_call(
    _mm_scale_body, grid=(GRID,),
    in_specs=[_dp_spec, _row_spec, _w_spec],
    out_specs=_row_spec, out_shape=_o_shape)

_mid = pl.pallas_call(
    _mid_body, grid=(GRID,),
    in_specs=[_dp_spec, _a_spec, _row_spec, _b_spec, _w_spec],
    out_specs=_row_spec, out_shape=_o_shape)

_out = pl.pallas_call(
    _out_body, grid=(GRID,),
    in_specs=[_dp_spec, _a_spec, _row_spec, _b_spec],
    out_specs=_row_spec, out_shape=_o_shape)


@jax.jit
def kernel(x, edge_index, W1, b1, W2, b2):
    src2 = edge_index[0].reshape(NW * ITERS, K)
    dst2 = edge_index[1].reshape(NW * ITERS, K)
    ones128 = jnp.ones((K, D), jnp.float32)
    z128 = jnp.zeros((ROWS, D), jnp.float32)

    dp = _deg_call(dst2, ones128, z128).reshape(NC, NPAD, D)
    y1 = _mm_scale(dp, x, W1)
    a1 = _edge_call(src2, dst2, y1, z128).reshape(NC, NPAD, D)
    y2 = _mid(dp, a1, y1, b1.reshape(1, D), W2)
    a2 = _edge_call(src2, dst2, y2, z128).reshape(NC, NPAD, D)
    return _out(dp, a2, y2, b2.reshape(1, D))


# speedup vs baseline: 22.7976x; 1.1940x over previous
import functools

import jax
import jax.numpy as jnp
from jax import lax
from jax.experimental import pallas as pl
from jax.experimental.pallas import tpu as pltpu
from jax.experimental.pallas import tpu_sc as plsc

N_NODES = 10000
N_EDGES = 320000
D = 128

NC = 2
NS = 16
NW = NC * NS
NPAD = 10240
ROWS = NPAD // NS
EPT = N_EDGES // NW
K = 80
ITERS = EPT // K

_mesh = plsc.VectorSubcoreMesh(core_axis_name="c", subcore_axis_name="s")


NBUF = 5
DGROUPS = ITERS // NBUF


def _deg_body(dst2, ones_hbm, z128, out_hbm, i0, i1, i2, i3, i4, ones_v, s0,
              s1, s2, s3, s4, shared):
    c = lax.axis_index("c")
    s = lax.axis_index("s")
    w = c * NS + s
    idx = [i0, i1, i2, i3, i4]
    sems = [s0, s1, s2, s3, s4]
    pltpu.sync_copy(ones_hbm, ones_v)
    pltpu.sync_copy(z128, shared.at[pl.ds(s * ROWS, ROWS)])
    plsc.subcore_barrier()

    for b in range(NBUF):
        pltpu.sync_copy(dst2.at[w * ITERS + b], idx[b])
        pltpu.async_copy(ones_v, shared.at[idx[b]], sems[b], add=True)

    def group(gi, carry):
        for b in range(NBUF):
            nxt = gi * NBUF + b + NBUF
            pltpu.make_async_copy(ones_v, shared.at[idx[b]], sems[b]).wait()
            pltpu.sync_copy(dst2.at[w * ITERS + nxt], idx[b])
            pltpu.async_copy(ones_v, shared.at[idx[b]], sems[b], add=True)
        return carry

    lax.fori_loop(0, DGROUPS - 1, group, 0)
    for b in range(NBUF):
        pltpu.make_async_copy(ones_v, shared.at[idx[b]], sems[b]).wait()
    plsc.subcore_barrier()
    pltpu.sync_copy(shared.at[pl.ds(s * ROWS, ROWS)],
                    out_hbm.at[pl.ds(c * NPAD + s * ROWS, ROWS)])


_deg_call = pl.kernel(
    _deg_body,
    out_type=jax.ShapeDtypeStruct((NC * NPAD, D), jnp.float32),
    mesh=_mesh,
    scratch_types=(
        [pltpu.VMEM((K,), jnp.int32)] * NBUF
        + [pltpu.VMEM((K, D), jnp.float32)]
        + [pltpu.SemaphoreType.DMA] * NBUF
        + [pltpu.VMEM_SHARED((NPAD, D), jnp.float32)]
    ),
)


def _edge_body(ei3, y_hbm, z128, out_hbm, x0, x1, r0, r1, g0, g1, shared):
    c = lax.axis_index("c")
    s = lax.axis_index("s")
    w = c * NS + s
    idx = [x0, x1]
    rows = [r0, r1]
    gsem = [g0, g1]
    pltpu.sync_copy(z128, shared.at[pl.ds(s * ROWS, ROWS)])
    plsc.subcore_barrier()

    for b in range(2):
        pltpu.sync_copy(ei3.at[w * ITERS + b], idx[b])
        pltpu.async_copy(y_hbm.at[idx[b].at[0]], rows[b], gsem[b])

    def group(gi, carry):
        for b in range(2):
            nxt = gi * 2 + b + 2
            pltpu.make_async_copy(y_hbm.at[idx[b].at[0]], rows[b],
                                  gsem[b]).wait()
            pltpu.sync_copy(rows[b], shared.at[idx[b].at[1]], add=True)
            pltpu.sync_copy(ei3.at[w * ITERS + nxt], idx[b])
            pltpu.async_copy(y_hbm.at[idx[b].at[0]], rows[b], gsem[b])
        return carry

    lax.fori_loop(0, (ITERS - 3) // 2, group, 0)
    pltpu.make_async_copy(y_hbm.at[idx[0].at[0]], rows[0], gsem[0]).wait()
    pltpu.sync_copy(rows[0], shared.at[idx[0].at[1]], add=True)
    pltpu.sync_copy(ei3.at[w * ITERS + ITERS - 1], idx[0])
    pltpu.async_copy(y_hbm.at[idx[0].at[0]], rows[0], gsem[0])
    pltpu.make_async_copy(y_hbm.at[idx[1].at[0]], rows[1], gsem[1]).wait()
    pltpu.sync_copy(rows[1], shared.at[idx[1].at[1]], add=True)
    pltpu.make_async_copy(y_hbm.at[idx[0].at[0]], rows[0], gsem[0]).wait()
    pltpu.sync_copy(rows[0], shared.at[idx[0].at[1]], add=True)

    plsc.subcore_barrier()
    pltpu.sync_copy(shared.at[pl.ds(s * ROWS, ROWS)],
                    out_hbm.at[pl.ds(c * NPAD + s * ROWS, ROWS)])


_edge_call = pl.kernel(
    _edge_body,
    out_type=jax.ShapeDtypeStruct((NC * NPAD, D), jnp.float32),
    mesh=_mesh,
    scratch_types=(
        [pltpu.VMEM((2, K), jnp.int32)] * 2
        + [pltpu.VMEM((K, D), jnp.float32)] * 2
        + [pltpu.SemaphoreType.DMA] * 2
        + [pltpu.VMEM_SHARED((NPAD, D), jnp.float32)]
    ),
)

BM = 1000
GRID = N_NODES // BM


def _dinv(dp_ref):
    return lax.rsqrt(1.0 + dp_ref[0, :, 0] + dp_ref[1, :, 0])


def _mm_scale_body(dp_ref, x_ref, w_ref, o_ref):
    dinv = _dinv(dp_ref)
    o_ref[...] = jnp.dot(x_ref[...], w_ref[...],
                         preferred_element_type=jnp.float32) * dinv[:, None]


def _mid_body(dp_ref, a_ref, y_ref, b_ref, w_ref, o_ref):
    dinv = _dinv(dp_ref)
    h = jnp.maximum(
        (a_ref[0] + a_ref[1] + y_ref[...]) * dinv[:, None] + b_ref[...], 0.0)
    o_ref[...] = jnp.dot(h, w_ref[...],
                         preferred_element_type=jnp.float32) * dinv[:, None]


def _out_body(dp_ref, a_ref, y_ref, b_ref, o_ref):
    dinv = _dinv(dp_ref)
    o_ref[...] = jnp.maximum(
        (a_ref[0] + a_ref[1] + y_ref[...]) * dinv[:, None] + b_ref[...], 0.0)


_dp_spec = pl.BlockSpec((2, BM, D), lambda i: (0, i, 0))
_a_spec = pl.BlockSpec((2, BM, D), lambda i: (0, i, 0))
_row_spec = pl.BlockSpec((BM, D), lambda i: (i, 0))
_w_spec = pl.BlockSpec((D, D), lambda i: (0, 0))
_b_spec = pl.BlockSpec((1, D), lambda i: (0, 0))
_o_shape = jax.ShapeDtypeStruct((N_NODES, D), jnp.float32)

_mm_scale = pl.pallas_call(
    _mm_scale_body, grid=(GRID,),
    in_specs=[_dp_spec, _row_spec, _w_spec],
    out_specs=_row_spec, out_shape=_o_shape)

_mid = pl.pallas_call(
    _mid_body, grid=(GRID,),
    in_specs=[_dp_spec, _a_spec, _row_spec, _b_spec, _w_spec],
    out_specs=_row_spec, out_shape=_o_shape)

_out = pl.pallas_call(
    _out_body, grid=(GRID,),
    in_specs=[_dp_spec, _a_spec, _row_spec, _b_spec],
    out_specs=_row_spec, out_shape=_o_shape)


@jax.jit
def kernel(x, edge_index, W1, b1, W2, b2):
    src2 = edge_index[0].reshape(NW * ITERS, K)
    dst2 = edge_index[1].reshape(NW * ITERS, K)
    ei3 = jnp.stack([src2, dst2], axis=1)
    ones128 = jnp.ones((K, D), jnp.float32)
    z128 = jnp.zeros((ROWS, D), jnp.float32)

    dp = _deg_call(dst2, ones128, z128).reshape(NC, NPAD, D)
    y1 = _mm_scale(dp, x, W1)
    a1 = _edge_call(ei3, y1, z128).reshape(NC, NPAD, D)
    y2 = _mid(dp, a1, y1, b1.reshape(1, D), W2)
    a2 = _edge_call(ei3, y2, z128).reshape(NC, NPAD, D)
    return _out(dp, a2, y2, b2.reshape(1, D))


# speedup vs baseline: 26.2287x; 1.1505x over previous
import functools

import jax
import jax.numpy as jnp
from jax import lax
from jax.experimental import pallas as pl
from jax.experimental.pallas import tpu as pltpu
from jax.experimental.pallas import tpu_sc as plsc

N_NODES = 10000
N_EDGES = 320000
D = 128

NC = 2
NS = 16
NW = NC * NS
NPAD = 10240
ROWS = NPAD // NS
EPT = N_EDGES // NW
K = 80
ITERS = EPT // K

_mesh = plsc.VectorSubcoreMesh(core_axis_name="c", subcore_axis_name="s")


NBUF = 5
DGROUPS = ITERS // NBUF


def _deg_body(dst2, ones_hbm, z128, out_hbm, i0, i1, i2, i3, i4, ones_v, s0,
              s1, s2, s3, s4, shared):
    c = lax.axis_index("c")
    s = lax.axis_index("s")
    w = c * NS + s
    idx = [i0, i1, i2, i3, i4]
    sems = [s0, s1, s2, s3, s4]
    pltpu.sync_copy(ones_hbm, ones_v)
    pltpu.sync_copy(z128, shared.at[pl.ds(s * ROWS, ROWS)])
    plsc.subcore_barrier()

    for b in range(NBUF):
        pltpu.sync_copy(dst2.at[w * ITERS + b], idx[b])
        pltpu.async_copy(ones_v, shared.at[idx[b]], sems[b], add=True)

    def group(gi, carry):
        for b in range(NBUF):
            nxt = gi * NBUF + b + NBUF
            pltpu.make_async_copy(ones_v, shared.at[idx[b]], sems[b]).wait()
            pltpu.sync_copy(dst2.at[w * ITERS + nxt], idx[b])
            pltpu.async_copy(ones_v, shared.at[idx[b]], sems[b], add=True)
        return carry

    lax.fori_loop(0, DGROUPS - 1, group, 0)
    for b in range(NBUF):
        pltpu.make_async_copy(ones_v, shared.at[idx[b]], sems[b]).wait()
    plsc.subcore_barrier()
    pltpu.sync_copy(shared.at[pl.ds(s * ROWS, ROWS)],
                    out_hbm.at[pl.ds(c * NPAD + s * ROWS, ROWS)])


_deg_call = pl.kernel(
    _deg_body,
    out_type=jax.ShapeDtypeStruct((NC * NPAD, D), jnp.float32),
    mesh=_mesh,
    scratch_types=(
        [pltpu.VMEM((K,), jnp.int32)] * NBUF
        + [pltpu.VMEM((K, D), jnp.float32)]
        + [pltpu.SemaphoreType.DMA] * NBUF
        + [pltpu.VMEM_SHARED((NPAD, D), jnp.float32)]
    ),
)


def _edge_body(ei3, y_hbm, z128, out_hbm, x00, x01, x10, x11, r0, r1, g0, g1,
               i00, i01, i10, i11, shared):
    c = lax.axis_index("c")
    s = lax.axis_index("s")
    w = c * NS + s
    base = w * ITERS
    idx = [[x00, x01], [x10, x11]]
    isem = [[i00, i01], [i10, i11]]
    rows = [r0, r1]
    gsem = [g0, g1]
    pltpu.sync_copy(z128, shared.at[pl.ds(s * ROWS, ROWS)])
    plsc.subcore_barrier()

    pltpu.sync_copy(ei3.at[base], idx[0][0])
    pltpu.sync_copy(ei3.at[base + 1], idx[1][0])
    pltpu.async_copy(ei3.at[base + 2], idx[0][1], isem[0][1])
    pltpu.async_copy(ei3.at[base + 3], idx[1][1], isem[1][1])
    for b in range(2):
        pltpu.async_copy(y_hbm.at[idx[b][0].at[0]], rows[b], gsem[b])

    def _step(ci, b, p, gather_next, refill):
        pltpu.make_async_copy(y_hbm.at[idx[b][p].at[0]], rows[b],
                              gsem[b]).wait()
        pltpu.sync_copy(rows[b], shared.at[idx[b][p].at[1]], add=True)
        if gather_next:
            pltpu.make_async_copy(ei3.at[base], idx[b][1 - p],
                                  isem[b][1 - p]).wait()
            pltpu.async_copy(y_hbm.at[idx[b][1 - p].at[0]], rows[b], gsem[b])
        if refill:
            pltpu.async_copy(ei3.at[base + ci + 4], idx[b][p], isem[b][p])

    def group(gi, carry):
        for k in range(4):
            ci = gi * 4 + k
            _step(ci, k % 2, (k // 2) % 2, True, True)
        return carry

    lax.fori_loop(0, (ITERS - 5) // 4, group, 0)
    t = ITERS - 5
    _step(t, 0, 0, True, False)
    pltpu.async_copy(ei3.at[base + ITERS - 1], idx[0][0], isem[0][0])
    _step(t + 1, 1, 0, True, False)
    _step(t + 2, 0, 1, True, False)
    _step(t + 3, 1, 1, False, False)
    _step(t + 4, 0, 0, False, False)

    plsc.subcore_barrier()
    pltpu.sync_copy(shared.at[pl.ds(s * ROWS, ROWS)],
                    out_hbm.at[pl.ds(c * NPAD + s * ROWS, ROWS)])


_edge_call = pl.kernel(
    _edge_body,
    out_type=jax.ShapeDtypeStruct((NC * NPAD, D), jnp.float32),
    mesh=_mesh,
    scratch_types=(
        [pltpu.VMEM((2, K), jnp.int32)] * 4
        + [pltpu.VMEM((K, D), jnp.float32)] * 2
        + [pltpu.SemaphoreType.DMA] * 2
        + [pltpu.SemaphoreType.DMA] * 4
        + [pltpu.VMEM_SHARED((NPAD, D), jnp.float32)]
    ),
)

BM = 1000
GRID = N_NODES // BM


def _dinv(dp_ref):
    return lax.rsqrt(1.0 + dp_ref[0, :, 0] + dp_ref[1, :, 0])


def _mm_scale_body(dp_ref, x_ref, w_ref, o_ref):
    dinv = _dinv(dp_ref)
    o_ref[...] = jnp.dot(x_ref[...], w_ref[...],
                         preferred_element_type=jnp.float32) * dinv[:, None]


def _mid_body(dp_ref, a_ref, y_ref, b_ref, w_ref, o_ref):
    dinv = _dinv(dp_ref)
    h = jnp.maximum(
        (a_ref[0] + a_ref[1] + y_ref[...]) * dinv[:, None] + b_ref[...], 0.0)
    o_ref[...] = jnp.dot(h, w_ref[...],
                         preferred_element_type=jnp.float32) * dinv[:, None]


def _out_body(dp_ref, a_ref, y_ref, b_ref, o_ref):
    dinv = _dinv(dp_ref)
    o_ref[...] = jnp.maximum(
        (a_ref[0] + a_ref[1] + y_ref[...]) * dinv[:, None] + b_ref[...], 0.0)


_dp_spec = pl.BlockSpec((2, BM, D), lambda i: (0, i, 0))
_a_spec = pl.BlockSpec((2, BM, D), lambda i: (0, i, 0))
_row_spec = pl.BlockSpec((BM, D), lambda i: (i, 0))
_w_spec = pl.BlockSpec((D, D), lambda i: (0, 0))
_b_spec = pl.BlockSpec((1, D), lambda i: (0, 0))
_o_shape = jax.ShapeDtypeStruct((N_NODES, D), jnp.float32)

_mm_scale = pl.pallas_call(
    _mm_scale_body, grid=(GRID,),
    in_specs=[_dp_spec, _row_spec, _w_spec],
    out_specs=_row_spec, out_shape=_o_shape)

_mid = pl.pallas_call(
    _mid_body, grid=(GRID,),
    in_specs=[_dp_spec, _a_spec, _row_spec, _b_spec, _w_spec],
    out_specs=_row_spec, out_shape=_o_shape)

_out = pl.pallas_call(
    _out_body, grid=(GRID,),
    in_specs=[_dp_spec, _a_spec, _row_spec, _b_spec],
    out_specs=_row_spec, out_shape=_o_shape)


@jax.jit
def kernel(x, edge_index, W1, b1, W2, b2):
    src2 = edge_index[0].reshape(NW * ITERS, K)
    dst2 = edge_index[1].reshape(NW * ITERS, K)
    ei3 = jnp.stack([src2, dst2], axis=1)
    ones128 = jnp.ones((K, D), jnp.float32)
    z128 = jnp.zeros((ROWS, D), jnp.float32)

    dp = _deg_call(dst2, ones128, z128).reshape(NC, NPAD, D)
    y1 = _mm_scale(dp, x, W1)
    a1 = _edge_call(ei3, y1, z128).reshape(NC, NPAD, D)
    y2 = _mid(dp, a1, y1, b1.reshape(1, D), W2)
    a2 = _edge_call(ei3, y2, z128).reshape(NC, NPAD, D)
    return _out(dp, a2, y2, b2.reshape(1, D))
